# baseline (speedup 1.0000x reference)
import functools

import jax
import jax.numpy as jnp
from jax import lax
from jax.experimental import pallas as pl
from jax.experimental.pallas import tpu as pltpu
from jax.experimental.pallas import tpu_sc as plsc

_C, _T, _H, _W = 3, 32, 224, 224
_S = _T // 4
_PLANE = _H * _W
_NC, _NS = 2, 16
_NW = _NC * _NS
_PARTS = 4
_CH = _PLANE // _PARTS
_CHUNKS = _C * _S * _PARTS
_PER_W = _CHUNKS // _NW


@functools.partial(
    pl.kernel,
    out_type=jax.ShapeDtypeStruct((_C * _S * _PLANE,), jnp.float32),
    mesh=plsc.VectorSubcoreMesh(core_axis_name="c", subcore_axis_name="s"),
    scratch_types=[
        [pltpu.VMEM((_CH,), jnp.float32) for _ in range(_PER_W)],
        pltpu.SemaphoreType.DMA,
    ],
)
def _sc_gather(frames_hbm, slow_hbm, bufs, sem):
    wid = lax.axis_index("s") * _NC + lax.axis_index("c")
    reads = []
    for k in range(_PER_W):
        g = wid * _PER_W + k
        plane = g // _PARTS
        part = g % _PARTS
        ch = plane // _S
        j = plane % _S
        t = (j * (_T - 1)) // (_S - 1)
        src = (ch * _T + t) * _PLANE + part * _CH
        reads.append(
            pltpu.async_copy(frames_hbm.at[pl.ds(src, _CH)], bufs[k], sem)
        )
    for d in reads:
        d.wait()
    writes = []
    for k in range(_PER_W):
        g = wid * _PER_W + k
        writes.append(
            pltpu.async_copy(bufs[k], slow_hbm.at[pl.ds(g * _CH, _CH)], sem)
        )
    for d in writes:
        d.wait()


def _copy_body(in_ref, out_ref):
    out_ref[...] = in_ref[...]


def kernel(frames):
    slow = _sc_gather(frames.reshape(-1))
    rows, lanes = _PLANE // 128, 128
    x = frames.reshape(_C * _T, rows, lanes)
    blk = 8
    fast = pl.pallas_call(
        _copy_body,
        grid=(_C * _T // blk,),
        in_specs=[pl.BlockSpec((blk, rows, lanes), lambda i: (i, 0, 0))],
        out_specs=pl.BlockSpec((blk, rows, lanes), lambda i: (i, 0, 0)),
        out_shape=jax.ShapeDtypeStruct((_C * _T, rows, lanes), frames.dtype),
    )(x)
    return slow.reshape(_C, _S, _H, _W), fast.reshape(_C, _T, _H, _W)


# speedup vs baseline: 2.9645x; 2.9645x over previous
import functools

import jax
import jax.numpy as jnp
from jax import lax
from jax.experimental import pallas as pl
from jax.experimental.pallas import tpu as pltpu
from jax.experimental.pallas import tpu_sc as plsc

_C, _T, _H, _W = 3, 32, 224, 224
_S = _T // 4
_NC, _NS = 2, 16
_NW = _NC * _NS
_NPLANES = _C * _S


@functools.partial(
    pl.kernel,
    out_type=jax.ShapeDtypeStruct((_NPLANES, _H, _W), jnp.float32),
    mesh=plsc.VectorSubcoreMesh(core_axis_name="c", subcore_axis_name="s"),
    scratch_types=[
        pltpu.VMEM((_H, _W), jnp.float32),
        pltpu.SemaphoreType.DMA,
    ],
)
def _sc_gather(frames_hbm, slow_hbm, buf, sem):
    wid = lax.axis_index("s") * _NC + lax.axis_index("c")

    @pl.when(wid < _NPLANES)
    def _():
        ch = wid // _S
        j = wid % _S
        t = (j * (_T - 1)) // (_S - 1)
        pltpu.async_copy(frames_hbm.at[ch * _T + t], buf, sem).wait()
        pltpu.async_copy(buf, slow_hbm.at[wid], sem).wait()


def kernel(frames):
    slow = _sc_gather(frames.reshape(_C * _T, _H, _W))
    return slow.reshape(_C, _S, _H, _W), frames


# speedup vs baseline: 2.9664x; 1.0007x over previous
import functools

import jax
import jax.numpy as jnp
from jax import lax
from jax.experimental import pallas as pl
from jax.experimental.pallas import tpu as pltpu
from jax.experimental.pallas import tpu_sc as plsc

_C, _T, _H, _W = 3, 32, 224, 224
_S = _T // 4
_NC, _NS = 2, 16
_NW = _NC * _NS
_NPLANES = _C * _S


@functools.partial(
    pl.kernel,
    out_type=jax.ShapeDtypeStruct((_NPLANES, _H, _W), jnp.float32),
    mesh=plsc.VectorSubcoreMesh(core_axis_name="c", subcore_axis_name="s"),
    scratch_types=[
        pltpu.VMEM((_H, _W), jnp.float32),
        pltpu.SemaphoreType.DMA,
    ],
)
def _sc_gather(frames_hbm, slow_hbm, buf, sem):
    wid = lax.axis_index("s") * _NC + lax.axis_index("c")

    @pl.when(wid < _NPLANES)
    def _():
        ch = wid // _S
        j = wid % _S
        t = (j * (_T - 1)) // (_S - 1)
        pltpu.async_copy(frames_hbm.at[ch * _T + t], buf, sem).wait()
        pltpu.async_copy(buf, slow_hbm.at[wid], sem).wait()


def kernel(frames):
    fast = jnp.copy(frames)
    slow = _sc_gather(frames.reshape(_C * _T, _H, _W))
    return slow.reshape(_C, _S, _H, _W), fast
